# baseline (speedup 1.0000x reference)
import functools
import math

import jax
import jax.numpy as jnp
from jax import lax
from jax.experimental import pallas as pl
from jax.experimental.pallas import tpu as pltpu
from jax.experimental.pallas import tpu_sc as plsc

VOCAB = 1_000_000
D = 64
B_ROWS = 4096
B_COLS = 200
B_TOTAL = B_ROWS * B_COLS

NC = 2
NS = 16
NW = NC * NS
PER_W = B_TOTAL // NW
CHUNK = 128
NG = PER_W // CHUNK
SCALE = math.sqrt(D)

_mesh = plsc.VectorSubcoreMesh(core_axis_name="c", subcore_axis_name="s")


@functools.partial(
    pl.kernel,
    out_type=jax.ShapeDtypeStruct((B_TOTAL, D), jnp.float32),
    mesh=_mesh,
    compiler_params=pltpu.CompilerParams(use_tc_tiling_on_sc=False),
    scratch_types=[
        pltpu.VMEM((NG, CHUNK), jnp.int32),
        pltpu.VMEM((CHUNK, D), jnp.float32),
        pltpu.SemaphoreType.DMA,
        pltpu.SemaphoreType.DMA,
    ],
)
def _emb_kernel(tokens_hbm, table_hbm, out_hbm, idx_v, rows_v, sem_g, sem_w):
    wid = lax.axis_index("s") * NC + lax.axis_index("c")
    base = wid * PER_W
    pltpu.sync_copy(tokens_hbm.at[wid], idx_v)

    def chunk_body(j, carry):
        pltpu.async_copy(table_hbm.at[idx_v.at[j]], rows_v, sem_g).wait()

        def scale_row(r, c):
            for k in range(D // 16):
                sl = pl.ds(k * 16, 16)
                rows_v[r, sl] = rows_v[r, sl] * SCALE
            return c

        lax.fori_loop(0, CHUNK, scale_row, 0, unroll=2)
        pltpu.async_copy(
            rows_v, out_hbm.at[pl.ds(base + j * CHUNK, CHUNK)], sem_w
        ).wait()
        return carry

    lax.fori_loop(0, NG, chunk_body, 0)


def kernel(tokens, table):
    idx = tokens.reshape(NW, NG, CHUNK)
    out = _emb_kernel(idx, table)
    return out.reshape(B_ROWS, B_COLS, D)
